# Initial kernel scaffold; baseline (speedup 1.0000x reference)
import jax
import jax.numpy as jnp
from jax.experimental import pallas as pl


def kernel(seq_len, embeddings):
    raise NotImplementedError("write your pallas kernel here")


# speedup vs baseline: 3.0236x; 3.0236x over previous
import functools

import jax
import jax.numpy as jnp
from jax import lax
from jax.experimental import pallas as pl
from jax.experimental.pallas import tpu as pltpu
from jax.experimental.pallas import tpu_sc as plsc

MAX_LEN = 2048
D_MODEL = 256
SEQ_LEN = 1024

_NUM_CORES = 2
_NUM_SUBCORES = 16
_NUM_WORKERS = _NUM_CORES * _NUM_SUBCORES
_ROWS_PER_WORKER = SEQ_LEN // _NUM_WORKERS
_STAGE_ROWS = 2 * SEQ_LEN
_STAGE_PER_SUBCORE = _STAGE_ROWS // _NUM_SUBCORES
_ROW_ELEMS = SEQ_LEN * D_MODEL


@functools.partial(
    pl.kernel,
    out_type=jax.ShapeDtypeStruct((SEQ_LEN * SEQ_LEN * D_MODEL,), jnp.float32),
    mesh=plsc.VectorSubcoreMesh(core_axis_name="c", subcore_axis_name="s"),
    scratch_types=[pltpu.VMEM_SHARED((_STAGE_ROWS * D_MODEL,), jnp.float32)],
)
def _rpe_copy(emb_hbm, out_hbm, staged):
    c = lax.axis_index("c")
    s = lax.axis_index("s")
    wid = c * _NUM_SUBCORES + s

    stripe = _STAGE_PER_SUBCORE * D_MODEL
    pltpu.sync_copy(
        emb_hbm.at[pl.ds((SEQ_LEN + s * _STAGE_PER_SUBCORE) * D_MODEL, stripe)],
        staged.at[pl.ds(s * stripe, stripe)],
    )
    plsc.subcore_barrier()

    def body(k, carry):
        i = wid * _ROWS_PER_WORKER + k
        pltpu.sync_copy(
            staged.at[pl.ds((SEQ_LEN - i) * D_MODEL, _ROW_ELEMS)],
            out_hbm.at[pl.ds(i * _ROW_ELEMS, _ROW_ELEMS)],
        )
        return carry

    lax.fori_loop(0, _ROWS_PER_WORKER, body, 0)


def kernel(seq_len, embeddings):
    del seq_len
    flat = _rpe_copy(embeddings.reshape(-1))
    return flat.reshape(SEQ_LEN, SEQ_LEN, D_MODEL)


# speedup vs baseline: 3.0321x; 1.0028x over previous
import functools

import jax
import jax.numpy as jnp
from jax import lax
from jax.experimental import pallas as pl
from jax.experimental.pallas import tpu as pltpu
from jax.experimental.pallas import tpu_sc as plsc

MAX_LEN = 2048
D_MODEL = 256
SEQ_LEN = 1024

_NUM_CORES = 2
_NUM_SUBCORES = 16
_NUM_WORKERS = _NUM_CORES * _NUM_SUBCORES
_ROWS_PER_WORKER = SEQ_LEN // _NUM_WORKERS
_STAGE_ROWS = 2 * SEQ_LEN
_STAGE_PER_SUBCORE = _STAGE_ROWS // _NUM_SUBCORES
_ROW_ELEMS = SEQ_LEN * D_MODEL


@functools.partial(
    pl.kernel,
    out_type=jax.ShapeDtypeStruct((SEQ_LEN * SEQ_LEN * D_MODEL,), jnp.float32),
    mesh=plsc.VectorSubcoreMesh(core_axis_name="c", subcore_axis_name="s"),
    scratch_types=[
        pltpu.VMEM_SHARED((_STAGE_ROWS * D_MODEL,), jnp.float32),
        pltpu.SemaphoreType.DMA,
    ],
)
def _rpe_copy(emb_hbm, out_hbm, staged, sem):
    c = lax.axis_index("c")
    s = lax.axis_index("s")
    wid = c * _NUM_SUBCORES + s

    stripe = _STAGE_PER_SUBCORE * D_MODEL
    pltpu.sync_copy(
        emb_hbm.at[pl.ds((SEQ_LEN + s * _STAGE_PER_SUBCORE) * D_MODEL, stripe)],
        staged.at[pl.ds(s * stripe, stripe)],
    )
    plsc.subcore_barrier()

    copies = []
    for k in range(_ROWS_PER_WORKER):
        i = wid * _ROWS_PER_WORKER + k
        copies.append(pltpu.async_copy(
            staged.at[pl.ds((SEQ_LEN - i) * D_MODEL, _ROW_ELEMS)],
            out_hbm.at[pl.ds(i * _ROW_ELEMS, _ROW_ELEMS)],
            sem,
        ))
    for cp in copies:
        cp.wait()


def kernel(seq_len, embeddings):
    del seq_len
    flat = _rpe_copy(embeddings.reshape(-1))
    return flat.reshape(SEQ_LEN, SEQ_LEN, D_MODEL)


# speedup vs baseline: 8.8598x; 2.9220x over previous
import functools

import jax
import jax.numpy as jnp
from jax import lax
from jax.experimental import pallas as pl
from jax.experimental.pallas import tpu as pltpu
from jax.experimental.pallas import tpu_sc as plsc

MAX_LEN = 2048
D_MODEL = 256
SEQ_LEN = 1024

_NUM_CORES = 2
_NUM_SUBCORES = 16
_RES = 8
_RES_PER_CORE = _RES // _NUM_CORES
_Q_PER_RES = SEQ_LEN // _RES
_SUB_PER_RES = _NUM_SUBCORES // _RES_PER_CORE
_Q_PER_WORKER = _Q_PER_RES // _SUB_PER_RES
_PLANE_ROWS = 2 * SEQ_LEN - _RES
_STRIPE = 512


@functools.partial(
    pl.kernel,
    out_type=jax.ShapeDtypeStruct((SEQ_LEN, SEQ_LEN, D_MODEL), jnp.float32),
    mesh=plsc.VectorSubcoreMesh(core_axis_name="c", subcore_axis_name="s"),
    scratch_types=[
        pltpu.VMEM_SHARED((_RES_PER_CORE, _PLANE_ROWS, D_MODEL), jnp.float32),
        pltpu.SemaphoreType.DMA,
    ],
)
def _rpe_copy(shifted_hbm, out_hbm, staged, sem):
    c = lax.axis_index("c")
    s = lax.axis_index("s")

    t_stage = s // _SUB_PER_RES
    off = jnp.minimum((s % _SUB_PER_RES) * _STRIPE, _PLANE_ROWS - _STRIPE)
    pltpu.sync_copy(
        shifted_hbm.at[c * _RES_PER_CORE + t_stage, pl.ds(off, _STRIPE), :],
        staged.at[t_stage, pl.ds(off, _STRIPE), :],
    )
    plsc.subcore_barrier()

    t = s % _RES_PER_CORE
    q0 = (s // _RES_PER_CORE) * _Q_PER_WORKER
    copies = []
    for m in range(_Q_PER_WORKER):
        q = q0 + m
        i = q * _RES + (c * _RES_PER_CORE + t)
        k0 = (SEQ_LEN - _RES) - q * _RES
        copies.append(pltpu.async_copy(
            staged.at[t, pl.ds(k0, SEQ_LEN), :],
            out_hbm.at[i],
            sem,
        ))
    for cp in copies:
        cp.wait()


def kernel(seq_len, embeddings):
    del seq_len
    shifted = jnp.stack([
        lax.slice(embeddings, (MAX_LEN // 2 + _RES - r, 0),
                  (MAX_LEN + SEQ_LEN - r, D_MODEL))
        for r in range(_RES)
    ])
    return _rpe_copy(shifted)
